# baseline (speedup 1.0000x reference)
import functools

import jax
import jax.numpy as jnp
from jax import lax
from jax.experimental import pallas as pl
from jax.experimental.pallas import tpu as pltpu
from jax.experimental.pallas import tpu_sc as plsc

NC = 2
NS = 16
DH = 128
DW = 16
K = 128
CPT = 80
ZR = 80


@functools.partial(jax.jit, static_argnums=(5,))
def _sc_segment_sum(xr, src2d, dst2d, z128, z16, n):
    npad = (n + ZR) // ZR * ZR
    nz = npad // ZR
    nw = n // ZR
    half = CPT // 2

    mesh = plsc.VectorSubcoreMesh(
        core_axis_name="c", subcore_axis_name="s", num_cores=NC,
        num_subcores=NS)

    @functools.partial(
        pl.kernel,
        out_type=(jax.ShapeDtypeStruct((NC, n, DH), jnp.float32),
                  jax.ShapeDtypeStruct((NC, n, DW), jnp.float32)),
        mesh=mesh,
        scratch_types=[
            pltpu.VMEM_SHARED((npad, DH), jnp.float32),
            pltpu.VMEM_SHARED((npad, DW), jnp.float32),
            (pltpu.VMEM((K,), jnp.int32),
             pltpu.VMEM((K,), jnp.int32)),
            (pltpu.VMEM((K,), jnp.int32),
             pltpu.VMEM((K,), jnp.int32)),
            (pltpu.VMEM((K, DH), jnp.float32),
             pltpu.VMEM((K, DH), jnp.float32)),
            pltpu.VMEM((K, DW), jnp.float32),
            (pltpu.SemaphoreType.DMA, pltpu.SemaphoreType.DMA),
            (pltpu.SemaphoreType.DMA, pltpu.SemaphoreType.DMA),
        ],
        compiler_params=pltpu.CompilerParams(use_tc_tiling_on_sc=False),
    )
    def body(xr_hbm, src_hbm, dst_hbm, z128_hbm, z16_hbm, s_out, d_out,
             s_sh, d_sh, sidx_v, didx_v, rows_v, ones_v, isem, gsem):
        cid = lax.axis_index("c")
        sid = lax.axis_index("s")

        one16 = jnp.ones((DW,), jnp.float32)

        def obody(i, _):
            ones_v[i, pl.ds(0, DW)] = one16
            return 0

        lax.fori_loop(0, K, obody, 0)

        def zbody(j, _):
            ch = sid + j * NS

            @pl.when(ch < nz)
            def _():
                pltpu.sync_copy(z128_hbm, s_sh.at[pl.ds(ch * ZR, ZR)])
                pltpu.sync_copy(z16_hbm, d_sh.at[pl.ds(ch * ZR, ZR)])
            return 0

        lax.fori_loop(0, (nz + NS - 1) // NS, zbody, 0)
        plsc.subcore_barrier()

        def start_idx(c, b):
            pltpu.async_copy(src_hbm.at[cid, sid, c], sidx_v[b], isem[b])
            pltpu.async_copy(dst_hbm.at[sid, c], didx_v[b], isem[b])

        def wait_idx(c, b):
            pltpu.make_async_copy(src_hbm.at[cid, sid, c], sidx_v[b],
                                  isem[b]).wait()
            pltpu.make_async_copy(dst_hbm.at[sid, c], didx_v[b],
                                  isem[b]).wait()

        def start_gather(b):
            pltpu.async_copy(xr_hbm.at[sidx_v[b]], rows_v[b], gsem[b])

        start_idx(0, 0)
        start_idx(1, 1)
        wait_idx(0, 0)
        start_gather(0)

        def ebody(jj, _):
            c0 = jj * 2
            for b in range(2):
                c = c0 + b

                @pl.when(c + 1 < CPT)
                def _():
                    wait_idx(c + 1, 1 - b)
                    start_gather(1 - b)

                pltpu.make_async_copy(xr_hbm.at[sidx_v[b]], rows_v[b],
                                      gsem[b]).wait()
                pltpu.sync_copy(rows_v[b], s_sh.at[didx_v[b]], add=True)
                mine = jnp.where(cid == 0, c < half, c >= half)

                @pl.when(mine)
                def _():
                    pltpu.sync_copy(ones_v, d_sh.at[didx_v[b]], add=True)

                @pl.when(c + 2 < CPT)
                def _():
                    start_idx(c + 2, b)
            return 0

        lax.fori_loop(0, CPT // 2, ebody, 0)
        plsc.subcore_barrier()

        def wbody(j, _):
            ch = sid + j * NS

            @pl.when(ch < nw)
            def _():
                pltpu.sync_copy(s_sh.at[pl.ds(ch * ZR, ZR)],
                                s_out.at[cid, pl.ds(ch * ZR, ZR)])
                pltpu.sync_copy(d_sh.at[pl.ds(ch * ZR, ZR)],
                                d_out.at[cid, pl.ds(ch * ZR, ZR)])
            return 0

        lax.fori_loop(0, (nw + NS - 1) // NS, wbody, 0)

    return body(xr, src2d, dst2d, z128, z16)


def _tc_combine(x, s2, d2, v4, b_sum):
    n, d = x.shape
    bn = 2000

    def body(x_ref, s_ref, d_ref, v_ref, b_ref, o_ref):
        s0 = s_ref[0]
        s1 = s_ref[1]
        xb = x_ref[...]
        deg = d_ref[0, :, 0:1] + d_ref[1, :, 0:1]
        acc = jnp.dot(xb[:, :DH] * s0, v_ref[0],
                      preferred_element_type=jnp.float32)
        acc += jnp.dot(xb[:, DH:] * s1, v_ref[1],
                       preferred_element_type=jnp.float32)
        acc += jnp.dot(s0, v_ref[2], preferred_element_type=jnp.float32)
        acc += jnp.dot(s1, v_ref[3], preferred_element_type=jnp.float32)
        acc += deg * b_ref[...]
        o_ref[...] = acc / jnp.maximum(deg, 1.0)

    return pl.pallas_call(
        body,
        grid=(n // bn,),
        in_specs=[
            pl.BlockSpec((bn, d), lambda i: (i, 0)),
            pl.BlockSpec((NC, bn, DH), lambda i: (0, i, 0)),
            pl.BlockSpec((NC, bn, DW), lambda i: (0, i, 0)),
            pl.BlockSpec((4, DH, d), lambda i: (0, 0, 0)),
            pl.BlockSpec((1, d), lambda i: (0, 0)),
        ],
        out_specs=pl.BlockSpec((bn, d), lambda i: (i, 0)),
        out_shape=jax.ShapeDtypeStruct((n, d), jnp.float32),
    )(x, s2, d2, v4, b_sum)


def kernel(x, edge_index, W_msg, b_msg, W_nb, b_nb):
    n, d = x.shape
    e = edge_index.shape[1]
    src = edge_index[0]
    dst = edge_index[1]

    epad = NS * CPT * K
    pad = epad - e
    srcp = jnp.concatenate([src, jnp.zeros((pad,), jnp.int32)])
    dstp = jnp.concatenate([dst, jnp.full((pad,), n, jnp.int32)])
    xr = x.reshape(2 * n, DH)
    src2d = jnp.stack([2 * srcp, 2 * srcp + 1]).reshape(NC, NS, CPT, K)
    dst2d = dstp.reshape(NS, CPT, K)
    z128 = jnp.zeros((ZR, DH), jnp.float32)
    z16 = jnp.zeros((ZR, DW), jnp.float32)

    s2, d2 = _sc_segment_sum(xr, src2d, dst2d, z128, z16, n)

    v4 = jnp.stack([W_msg[:, :DH].T, W_msg[:, DH:].T,
                    W_nb[:, :DH].T, W_nb[:, DH:].T])
    b_sum = (b_msg + b_nb).reshape(1, d)
    return _tc_combine(x, s2, d2, v4, b_sum)


# speedup vs baseline: 1.0274x; 1.0274x over previous
import functools

import jax
import jax.numpy as jnp
from jax import lax
from jax.experimental import pallas as pl
from jax.experimental.pallas import tpu as pltpu
from jax.experimental.pallas import tpu_sc as plsc

NC = 2
NS = 16
DH = 128
DW = 16
K = 128
CPT = 80
ZR = 80
NBUF = 2


@functools.partial(jax.jit, static_argnums=(5,))
def _sc_segment_sum(xr, src2d, dst2d, z128, z16, n):
    npad = (n + ZR) // ZR * ZR
    nz = npad // ZR
    nw = n // ZR
    half = CPT // 2

    mesh = plsc.VectorSubcoreMesh(
        core_axis_name="c", subcore_axis_name="s", num_cores=NC,
        num_subcores=NS)

    @functools.partial(
        pl.kernel,
        out_type=(jax.ShapeDtypeStruct((NC, n, DH), jnp.float32),
                  jax.ShapeDtypeStruct((NC, n, DW), jnp.float32)),
        mesh=mesh,
        scratch_types=[
            pltpu.VMEM_SHARED((npad, DH), jnp.float32),
            pltpu.VMEM_SHARED((npad, DW), jnp.float32),
            tuple(pltpu.VMEM((K,), jnp.int32)
                  for _ in range(4)),
            tuple(pltpu.VMEM((K,), jnp.int32)
                  for _ in range(4)),
            tuple(pltpu.VMEM((K, DH), jnp.float32)
                  for _ in range(NBUF)),
            pltpu.VMEM((K, DW), jnp.float32),
            tuple(pltpu.SemaphoreType.DMA for _ in range(4)),
            tuple(pltpu.SemaphoreType.DMA for _ in range(NBUF)),
            tuple(pltpu.SemaphoreType.DMA for _ in range(NBUF)),
            pltpu.SemaphoreType.DMA,
        ],
        compiler_params=pltpu.CompilerParams(use_tc_tiling_on_sc=False),
    )
    def body(xr_hbm, src_hbm, dst_hbm, z128_hbm, z16_hbm, s_out, d_out,
             s_sh, d_sh, sidx_v, didx_v, rows, ones_v, isem, gsem, ssem,
             dsem):
        cid = lax.axis_index("c")
        sid = lax.axis_index("s")

        one16 = jnp.ones((DW,), jnp.float32)

        def obody(i, _):
            ones_v[i, pl.ds(0, DW)] = one16
            return 0

        lax.fori_loop(0, K, obody, 0)

        def start_idx(c, ib):
            pltpu.async_copy(src_hbm.at[cid, sid, c], sidx_v[ib], isem[ib])
            pltpu.async_copy(dst_hbm.at[sid, c], didx_v[ib], isem[ib])

        def wait_idx(c, ib):
            pltpu.make_async_copy(src_hbm.at[cid, sid, c], sidx_v[ib],
                                  isem[ib]).wait()
            pltpu.make_async_copy(dst_hbm.at[sid, c], didx_v[ib],
                                  isem[ib]).wait()

        def start_gather(ib, b):
            pltpu.async_copy(xr_hbm.at[sidx_v[ib]], rows[b], gsem[b])

        def wait_gather(ib, b):
            pltpu.make_async_copy(xr_hbm.at[sidx_v[ib]], rows[b],
                                  gsem[b]).wait()

        start_idx(0, 0)
        start_idx(1, 1)
        wait_idx(0, 0)
        start_gather(0, 0)

        def zbody(j, _):
            ch = sid + j * NS

            @pl.when(ch < nz)
            def _():
                pltpu.sync_copy(z128_hbm, s_sh.at[pl.ds(ch * ZR, ZR)])
                pltpu.sync_copy(z16_hbm, d_sh.at[pl.ds(ch * ZR, ZR)])
            return 0

        lax.fori_loop(0, (nz + NS - 1) // NS, zbody, 0)
        plsc.subcore_barrier()

        start_idx(2, 2)
        start_idx(3, 3)
        wait_idx(1, 1)
        start_gather(1, 1)

        first_mine = jnp.where(cid == 0, 0, half)
        last_mine = jnp.where(cid == 0, half - 1, CPT - 1)

        def ebody(blk, _):
            for j in range(4):
                c = blk * 4 + j
                b = j % NBUF
                pltpu.make_async_copy(xr_hbm.at[sidx_v[j]], rows[b],
                                      gsem[b]).wait()
                pltpu.async_copy(rows[b], s_sh.at[didx_v[j]], ssem[b],
                                 add=True)
                mine = jnp.where(cid == 0, c < half, c >= half)

                @pl.when(mine & (c != first_mine))
                def _():
                    pltpu.make_async_copy(ones_v, d_sh.at[didx_v[j]],
                                          dsem).wait()

                @pl.when(mine)
                def _():
                    pltpu.async_copy(ones_v, d_sh.at[didx_v[j]], dsem,
                                     add=True)

                @pl.when(mine & (c == last_mine))
                def _():
                    pltpu.make_async_copy(ones_v, d_sh.at[didx_v[j]],
                                          dsem).wait()

                @pl.when(c + 2 < CPT)
                def _():
                    pltpu.make_async_copy(rows[b], s_sh.at[didx_v[j]],
                                          ssem[b]).wait()

                    @pl.when(c + 4 < CPT)
                    def _():
                        start_idx(c + 4, j)

                    wait_idx(c + 2, (j + 2) % 4)
                    pltpu.async_copy(xr_hbm.at[sidx_v[(j + 2) % 4]],
                                     rows[b], gsem[b])
            return 0

        lax.fori_loop(0, CPT // 4, ebody, 0)
        pltpu.make_async_copy(rows[0], s_sh.at[didx_v[2]], ssem[0]).wait()
        pltpu.make_async_copy(rows[1], s_sh.at[didx_v[3]], ssem[1]).wait()
        plsc.subcore_barrier()

        def wbody(j, _):
            ch = sid + j * NS

            @pl.when(ch < nw)
            def _():
                pltpu.sync_copy(s_sh.at[pl.ds(ch * ZR, ZR)],
                                s_out.at[cid, pl.ds(ch * ZR, ZR)])
                pltpu.sync_copy(d_sh.at[pl.ds(ch * ZR, ZR)],
                                d_out.at[cid, pl.ds(ch * ZR, ZR)])
            return 0

        lax.fori_loop(0, (nw + NS - 1) // NS, wbody, 0)

    return body(xr, src2d, dst2d, z128, z16)


def _tc_combine(x, s2, d2, v4, b_sum):
    n, d = x.shape
    bn = 2000

    def body(x_ref, s_ref, d_ref, v_ref, b_ref, o_ref):
        s0 = s_ref[0]
        s1 = s_ref[1]
        xb = x_ref[...]
        deg = d_ref[0, :, 0:1] + d_ref[1, :, 0:1]
        acc = jnp.dot(xb[:, :DH] * s0, v_ref[0],
                      preferred_element_type=jnp.float32)
        acc += jnp.dot(xb[:, DH:] * s1, v_ref[1],
                       preferred_element_type=jnp.float32)
        acc += jnp.dot(s0, v_ref[2], preferred_element_type=jnp.float32)
        acc += jnp.dot(s1, v_ref[3], preferred_element_type=jnp.float32)
        acc += deg * b_ref[...]
        o_ref[...] = acc / jnp.maximum(deg, 1.0)

    return pl.pallas_call(
        body,
        grid=(n // bn,),
        in_specs=[
            pl.BlockSpec((bn, d), lambda i: (i, 0)),
            pl.BlockSpec((NC, bn, DH), lambda i: (0, i, 0)),
            pl.BlockSpec((NC, bn, DW), lambda i: (0, i, 0)),
            pl.BlockSpec((4, DH, d), lambda i: (0, 0, 0)),
            pl.BlockSpec((1, d), lambda i: (0, 0)),
        ],
        out_specs=pl.BlockSpec((bn, d), lambda i: (i, 0)),
        out_shape=jax.ShapeDtypeStruct((n, d), jnp.float32),
    )(x, s2, d2, v4, b_sum)


def kernel(x, edge_index, W_msg, b_msg, W_nb, b_nb):
    n, d = x.shape
    e = edge_index.shape[1]
    src = edge_index[0]
    dst = edge_index[1]

    epad = NS * CPT * K
    pad = epad - e
    srcp = jnp.concatenate([src, jnp.zeros((pad,), jnp.int32)])
    dstp = jnp.concatenate([dst, jnp.full((pad,), n, jnp.int32)])
    xr = x.reshape(2 * n, DH)
    src2d = jnp.stack([2 * srcp, 2 * srcp + 1]).reshape(NC, NS, CPT, K)
    dst2d = dstp.reshape(NS, CPT, K)
    z128 = jnp.zeros((ZR, DH), jnp.float32)
    z16 = jnp.zeros((ZR, DW), jnp.float32)

    s2, d2 = _sc_segment_sum(xr, src2d, dst2d, z128, z16, n)

    v4 = jnp.stack([W_msg[:, :DH].T, W_msg[:, DH:].T,
                    W_nb[:, :DH].T, W_nb[:, DH:].T])
    b_sum = (b_msg + b_nb).reshape(1, d)
    return _tc_combine(x, s2, d2, v4, b_sum)
